# Initial kernel scaffold; baseline (speedup 1.0000x reference)
import jax
import jax.numpy as jnp
from jax.experimental import pallas as pl


def kernel(x, codebook):
    raise NotImplementedError("write your pallas kernel here")


# speedup vs baseline: 1.0345x; 1.0345x over previous
import functools

import jax
import jax.numpy as jnp
from jax import lax
from jax.experimental import pallas as pl
from jax.experimental.pallas import tpu as pltpu
from jax.experimental.pallas import tpu_sc as plsc

_COMMITMENT = 0.25

_TM = 256
_NC = 2048

_INTERPRET = False


def _argmin_body(x_ref, cbt_ref, idx_ref, dpart_ref, cbtb_ref, cbn_ref):
    i = pl.program_id(0)
    D, CB = cbt_ref.shape
    nchunks = CB // _NC

    @pl.when(i == 0)
    def _init():
        for c in range(nchunks):
            sl = pl.ds(c * _NC, _NC)
            ch = cbt_ref[:, sl]
            cbtb_ref[:, sl] = ch.astype(jnp.bfloat16)
            cbn_ref[0:1, sl] = jnp.sum(ch * ch, axis=0, keepdims=True)

    xf = x_ref[...]
    xb = xf.astype(jnp.bfloat16)
    xn = jnp.sum(xf * xf, axis=1, keepdims=True)

    m_run = jnp.full((_TM, 1), jnp.inf, jnp.float32)
    i_run = jnp.zeros((_TM, 1), jnp.int32)
    for c in range(nchunks):
        sl = pl.ds(c * _NC, _NC)
        s = lax.dot_general(xb, cbtb_ref[:, sl], (((1,), (0,)), ((), ())),
                            preferred_element_type=jnp.float32)
        d = (xn - 2.0 * s) + cbn_ref[0:1, sl]
        mc = jnp.min(d, axis=1, keepdims=True)
        io = lax.broadcasted_iota(jnp.int32, (_TM, _NC), 1) + (c * _NC)
        ic = jnp.min(jnp.where(d == mc, io, CB), axis=1, keepdims=True)
        upd = mc < m_run
        i_run = jnp.where(upd, ic, i_run)
        m_run = jnp.where(upd, mc, m_run).astype(jnp.bfloat16).astype(jnp.float32)

    idx_ref[...] = i_run
    p = jnp.sum(m_run)
    dpart_ref[...] = jnp.full((1, 1, 128), p * (1.0 / 128.0), jnp.float32)


def _argmin_call(flat_f32, cbt_f32):
    NT, D = flat_f32.shape
    CB = cbt_f32.shape[1]
    G = NT // _TM
    return pl.pallas_call(
        _argmin_body,
        grid=(G,),
        in_specs=[
            pl.BlockSpec((_TM, D), lambda i: (i, 0)),
            pl.BlockSpec((D, CB), lambda i: (0, 0)),
        ],
        out_specs=[
            pl.BlockSpec((_TM, 1), lambda i: (i, 0)),
            pl.BlockSpec((1, 1, 128), lambda i: (i, 0, 0)),
        ],
        out_shape=[
            jax.ShapeDtypeStruct((NT, 1), jnp.int32),
            jax.ShapeDtypeStruct((G, 1, 128), jnp.float32),
        ],
        scratch_shapes=[
            pltpu.VMEM((D, CB), jnp.bfloat16),
            pltpu.VMEM((1, CB), jnp.float32),
        ],
        interpret=_INTERPRET,
    )(flat_f32, cbt_f32)


def _gather_call(codebook, idx_flat):
    B = idx_flat.shape[0]
    V, D = codebook.shape
    info = plsc.get_sparse_core_info()
    nw = info.num_cores * info.num_subcores
    b_per_w = B // nw
    mesh = plsc.VectorSubcoreMesh(core_axis_name="c", subcore_axis_name="s")

    @functools.partial(
        pl.kernel, mesh=mesh,
        out_type=jax.ShapeDtypeStruct((B, D), jnp.float32),
        scratch_types=[
            pltpu.VMEM((b_per_w,), jnp.int32),
            pltpu.VMEM((b_per_w, D), jnp.float32),
            pltpu.SemaphoreType.DMA,
        ],
    )
    def k(table_hbm, idx_hbm, out_hbm, idx_v, rows_v, sem):
        wid = lax.axis_index("s") * info.num_cores + lax.axis_index("c")
        base = wid * b_per_w
        pltpu.sync_copy(idx_hbm.at[pl.ds(base, b_per_w)], idx_v)
        pltpu.async_copy(table_hbm.at[idx_v], rows_v, sem).wait()
        pltpu.sync_copy(rows_v, out_hbm.at[pl.ds(base, b_per_w)])

    return k(codebook, idx_flat)


def kernel(x, codebook):
    B, N, D = x.shape
    NT = B * N
    flat = x.reshape(NT, D)
    cbt = jnp.swapaxes(codebook, 0, 1)
    idx2, dpart = _argmin_call(flat, cbt)
    idx_flat = idx2.reshape(NT)
    indices = idx2.reshape(B, N)
    q = _gather_call(codebook, idx_flat)
    quantized_x = q.reshape(B, N, D)
    commit_loss = jnp.sum(dpart) * (_COMMITMENT / (NT * D))
    return quantized_x, commit_loss, indices


# speedup vs baseline: 1.0911x; 1.0547x over previous
import functools

import jax
import jax.numpy as jnp
from jax import lax
from jax.experimental import pallas as pl
from jax.experimental.pallas import tpu as pltpu
from jax.experimental.pallas import tpu_sc as plsc

_COMMITMENT = 0.25

_TM = 256
_NC = 2048

_INTERPRET = False


def _argmin_body(x_ref, cbt_ref, idx_ref, dpart_ref, cbtb_ref, cbn_ref):
    i = pl.program_id(0)
    D, CB = cbt_ref.shape
    nchunks = CB // _NC

    @pl.when(i == 0)
    def _init():
        for c in range(nchunks):
            sl = pl.ds(c * _NC, _NC)
            ch = cbt_ref[:, sl]
            cbtb_ref[:, sl] = ch.astype(jnp.bfloat16)
            cbn_ref[0:1, sl] = jnp.sum(ch * ch, axis=0, keepdims=True)

    xf = x_ref[...]
    xb = (-2.0 * xf).astype(jnp.bfloat16)
    xn = jnp.sum(xf * xf, axis=1, keepdims=True)

    io = lax.broadcasted_iota(jnp.int32, (_TM, _NC), 1)
    m_run = jnp.full((_TM, 1), jnp.inf, jnp.float32)
    i_run = jnp.zeros((_TM, 1), jnp.int32)
    for c in range(nchunks):
        sl = pl.ds(c * _NC, _NC)
        s = lax.dot_general(xb, cbtb_ref[:, sl], (((1,), (0,)), ((), ())),
                            preferred_element_type=jnp.float32)
        d = (xn + s) + cbn_ref[0:1, sl]
        mc = jnp.min(d, axis=1, keepdims=True)
        ic = jnp.min(jnp.where(d == mc, io, _NC), axis=1, keepdims=True) + (c * _NC)
        upd = mc < m_run
        i_run = jnp.where(upd, ic, i_run)
        m_run = jnp.where(upd, mc, m_run).astype(jnp.bfloat16).astype(jnp.float32)

    idx_ref[...] = i_run
    p = jnp.sum(m_run)
    dpart_ref[...] = jnp.full((1, 1, 128), p * (1.0 / 128.0), jnp.float32)


def _argmin_call(flat_f32, cbt_f32):
    NT, D = flat_f32.shape
    CB = cbt_f32.shape[1]
    G = NT // _TM
    return pl.pallas_call(
        _argmin_body,
        grid=(G,),
        in_specs=[
            pl.BlockSpec((_TM, D), lambda i: (i, 0)),
            pl.BlockSpec((D, CB), lambda i: (0, 0)),
        ],
        out_specs=[
            pl.BlockSpec((_TM, 1), lambda i: (i, 0)),
            pl.BlockSpec((1, 1, 128), lambda i: (i, 0, 0)),
        ],
        out_shape=[
            jax.ShapeDtypeStruct((NT, 1), jnp.int32),
            jax.ShapeDtypeStruct((G, 1, 128), jnp.float32),
        ],
        scratch_shapes=[
            pltpu.VMEM((D, CB), jnp.bfloat16),
            pltpu.VMEM((1, CB), jnp.float32),
        ],
        interpret=_INTERPRET,
    )(flat_f32, cbt_f32)


def _gather_call(codebook, idx_flat):
    B = idx_flat.shape[0]
    V, D = codebook.shape
    info = plsc.get_sparse_core_info()
    nw = info.num_cores * info.num_subcores
    b_per_w = B // nw
    mesh = plsc.VectorSubcoreMesh(core_axis_name="c", subcore_axis_name="s")

    @functools.partial(
        pl.kernel, mesh=mesh,
        out_type=jax.ShapeDtypeStruct((B, D), jnp.float32),
        scratch_types=[
            pltpu.VMEM((b_per_w,), jnp.int32),
            pltpu.VMEM((b_per_w, D), jnp.float32),
            pltpu.SemaphoreType.DMA,
        ],
    )
    def k(table_hbm, idx_hbm, out_hbm, idx_v, rows_v, sem):
        wid = lax.axis_index("s") * info.num_cores + lax.axis_index("c")
        base = wid * b_per_w
        pltpu.sync_copy(idx_hbm.at[pl.ds(base, b_per_w)], idx_v)
        pltpu.async_copy(table_hbm.at[idx_v], rows_v, sem).wait()
        pltpu.sync_copy(rows_v, out_hbm.at[pl.ds(base, b_per_w)])

    return k(codebook, idx_flat)


def kernel(x, codebook):
    B, N, D = x.shape
    NT = B * N
    flat = x.reshape(NT, D)
    cbt = jnp.swapaxes(codebook, 0, 1)
    idx2, dpart = _argmin_call(flat, cbt)
    idx_flat = idx2.reshape(NT)
    indices = idx2.reshape(B, N)
    q = _gather_call(codebook, idx_flat)
    quantized_x = q.reshape(B, N, D)
    commit_loss = jnp.sum(dpart) * (_COMMITMENT / (NT * D))
    return quantized_x, commit_loss, indices


# speedup vs baseline: 1.1323x; 1.0378x over previous
import functools

import jax
import jax.numpy as jnp
from jax import lax
from jax.experimental import pallas as pl
from jax.experimental.pallas import tpu as pltpu
from jax.experimental.pallas import tpu_sc as plsc

_COMMITMENT = 0.25

_TM = 256
_NC = 2048

_INTERPRET = False


def _argmin_body(x_ref, cbt_ref, idx_ref, dpart_ref, cbtb_ref, cbn_ref):
    i = pl.program_id(0)
    D, CB = cbt_ref.shape
    nchunks = CB // _NC

    @pl.when(i == 0)
    def _init():
        for c in range(nchunks):
            sl = pl.ds(c * _NC, _NC)
            ch = cbt_ref[:, sl]
            cbtb_ref[:, sl] = ch.astype(jnp.bfloat16)
            cbn_ref[0:1, sl] = jnp.sum(ch * ch, axis=0, keepdims=True)

    xf = x_ref[...]
    xb = (-2.0 * xf).astype(jnp.bfloat16)
    xn = jnp.sum(xf * xf, axis=1, keepdims=True)

    io = lax.broadcasted_iota(jnp.int32, (_TM, _NC), 1)
    m_run = jnp.full((_TM, 1), jnp.inf, jnp.float32)
    i_run = jnp.zeros((_TM, 1), jnp.int32)
    for c in range(nchunks):
        sl = pl.ds(c * _NC, _NC)
        s = lax.dot_general(xb, cbtb_ref[:, sl], (((1,), (0,)), ((), ())),
                            preferred_element_type=jnp.float32)
        d = (xn + s) + cbn_ref[0:1, sl]
        mc = jnp.min(d, axis=1, keepdims=True)
        ic = jnp.argmin(d, axis=1).astype(jnp.int32)[:, None] + (c * _NC)
        upd = mc < m_run
        i_run = jnp.where(upd, ic, i_run)
        m_run = jnp.where(upd, mc, m_run).astype(jnp.bfloat16).astype(jnp.float32)

    idx_ref[...] = i_run
    p = jnp.sum(m_run)
    dpart_ref[...] = jnp.full((1, 1, 128), p * (1.0 / 128.0), jnp.float32)


def _argmin_call(flat_f32, cbt_f32):
    NT, D = flat_f32.shape
    CB = cbt_f32.shape[1]
    G = NT // _TM
    return pl.pallas_call(
        _argmin_body,
        grid=(G,),
        in_specs=[
            pl.BlockSpec((_TM, D), lambda i: (i, 0)),
            pl.BlockSpec((D, CB), lambda i: (0, 0)),
        ],
        out_specs=[
            pl.BlockSpec((_TM, 1), lambda i: (i, 0)),
            pl.BlockSpec((1, 1, 128), lambda i: (i, 0, 0)),
        ],
        out_shape=[
            jax.ShapeDtypeStruct((NT, 1), jnp.int32),
            jax.ShapeDtypeStruct((G, 1, 128), jnp.float32),
        ],
        scratch_shapes=[
            pltpu.VMEM((D, CB), jnp.bfloat16),
            pltpu.VMEM((1, CB), jnp.float32),
        ],
        interpret=_INTERPRET,
    )(flat_f32, cbt_f32)


def _gather_call(codebook, idx_flat):
    B = idx_flat.shape[0]
    V, D = codebook.shape
    info = plsc.get_sparse_core_info()
    nw = info.num_cores * info.num_subcores
    b_per_w = B // nw
    mesh = plsc.VectorSubcoreMesh(core_axis_name="c", subcore_axis_name="s")

    @functools.partial(
        pl.kernel, mesh=mesh,
        out_type=jax.ShapeDtypeStruct((B, D), jnp.float32),
        scratch_types=[
            pltpu.VMEM((b_per_w,), jnp.int32),
            pltpu.VMEM((b_per_w, D), jnp.float32),
            pltpu.SemaphoreType.DMA,
        ],
    )
    def k(table_hbm, idx_hbm, out_hbm, idx_v, rows_v, sem):
        wid = lax.axis_index("s") * info.num_cores + lax.axis_index("c")
        base = wid * b_per_w
        pltpu.sync_copy(idx_hbm.at[pl.ds(base, b_per_w)], idx_v)
        pltpu.async_copy(table_hbm.at[idx_v], rows_v, sem).wait()
        pltpu.sync_copy(rows_v, out_hbm.at[pl.ds(base, b_per_w)])

    return k(codebook, idx_flat)


def kernel(x, codebook):
    B, N, D = x.shape
    NT = B * N
    flat = x.reshape(NT, D)
    cbt = jnp.swapaxes(codebook, 0, 1)
    idx2, dpart = _argmin_call(flat, cbt)
    idx_flat = idx2.reshape(NT)
    indices = idx2.reshape(B, N)
    q = _gather_call(codebook, idx_flat)
    quantized_x = q.reshape(B, N, D)
    commit_loss = jnp.sum(dpart) * (_COMMITMENT / (NT * D))
    return quantized_x, commit_loss, indices
